# baseline (speedup 1.0000x reference)
import jax
import jax.numpy as jnp
from jax.experimental import pallas as pl
from jax.experimental.pallas import tpu as pltpu


def _make_body(sub):
    def body(x_ref, xn_ref, wih_ref, whh_ref, wtag_ref, btag_ref, out_ref,
             h_ref, c_ref, gx_ref, hs_ref):
        TC, _, B, E = x_ref.shape
        H = whh_ref.shape[0]
        TP = wtag_ref.shape[1]
        NS = TC // sub

        @pl.when(pl.program_id(1) == 0)
        def _reset_state():
            h_ref[...] = jnp.zeros_like(h_ref)
            c_ref[...] = jnp.zeros_like(c_ref)

        def project_from(src_ref, j):
            xs = src_ref[pl.ds(j * sub, sub)].reshape(sub, B, E)
            chan = jax.lax.broadcasted_iota(jnp.int32, xs.shape, 2)
            xs = jnp.where(chan < 20, xs, jax.nn.sigmoid(xs)).astype(
                jnp.bfloat16)
            ones = jnp.ones((sub, B, 8), jnp.bfloat16)
            xs = jnp.concatenate([xs, ones], axis=2)
            gx_ref[pl.ds(j * sub * B, sub * B), :] = jnp.dot(
                xs.reshape(sub * B, E + 8), wih_ref[...],
                preferred_element_type=jnp.float32)

        def project_inputs(j):
            project_from(x_ref, j)

        def project_tags(j):
            tags = (jnp.dot(hs_ref[pl.ds(j * sub * B, sub * B), :],
                            wtag_ref[...],
                            preferred_element_type=jnp.float32)
                    + btag_ref[...])
            out_ref[pl.ds(j * sub, sub)] = tags.reshape(sub, 1, B, TP)

        whh = whh_ref[...]
        h = h_ref[...]
        c = c_ref[...]

        @pl.when(pl.program_id(1) == 0)
        def _prologue():
            project_inputs(0)

        for j in range(NS):
            if j + 1 < NS:
                project_inputs(j + 1)
            for ts in range(sub):
                t = j * sub + ts
                z = gx_ref[pl.ds(t * B, B), :] + jnp.dot(
                    h, whh, preferred_element_type=jnp.float32)
                th = jnp.tanh(z)
                ti = th[:, 0 * H:1 * H]
                tf = th[:, 1 * H:2 * H]
                tg = th[:, 2 * H:3 * H]
                to = th[:, 3 * H:4 * H]
                c = ((tf + 1.0) * c + (ti + 1.0) * tg) * 0.5
                h = ((to + 1.0) * (0.5 * jnp.tanh(c))).astype(jnp.bfloat16)
                hs_ref[pl.ds(t * B, B), :] = h
            if j >= 1:
                project_tags(j - 1)

        project_from(xn_ref, 0)
        project_tags(NS - 1)

        h_ref[...] = h
        c_ref[...] = c

    return body


def kernel(x, w_ih, w_hh, b_ih, b_hh, w_tag, b_tag):
    S, B, E = x.shape
    H = w_ih.shape[0] // 4
    T = w_tag.shape[0]

    B_pad = -(-B // 8) * 8
    T_pad = -(-T // 128) * 128

    b_block = B_pad
    NB = 1

    t_chunk = min(64, S)
    while S % t_chunk:
        t_chunk //= 2
    while t_chunk > 8 and t_chunk * b_block * 4 * H * 4 > 18 * 1024 * 1024:
        t_chunk //= 2
    sub = min(8, t_chunk)
    S_pad = pl.cdiv(S, t_chunk) * t_chunk

    x_p = jnp.pad(x, ((0, S_pad - S), (0, B_pad - B), (0, 0)))
    x_v = x_p.reshape(S_pad, NB, b_block, E)

    gcol = jnp.arange(4 * H)
    gate_scale = jnp.where((gcol >= 2 * H) & (gcol < 3 * H), 1.0, 0.5)

    wih_t = jnp.concatenate(
        [jnp.transpose(w_ih) * gate_scale,
         ((b_ih + b_hh) * gate_scale).reshape(1, 4 * H),
         jnp.zeros((7, 4 * H), jnp.float32)], axis=0).astype(jnp.bfloat16)
    whh_t = (jnp.transpose(w_hh) * gate_scale).astype(jnp.bfloat16)
    wtag_t = jnp.pad(jnp.transpose(w_tag).astype(jnp.bfloat16),
                     ((0, 0), (0, T_pad - T)))
    btag = jnp.pad(b_tag, (0, T_pad - T)).reshape(1, T_pad)

    nt = S_pad // t_chunk
    out = pl.pallas_call(
        _make_body(sub),
        out_shape=jax.ShapeDtypeStruct((S_pad, NB, b_block, T_pad),
                                       jnp.float32),
        grid=(NB, nt),
        in_specs=[
            pl.BlockSpec((t_chunk, 1, b_block, E), lambda b, t: (t, b, 0, 0)),
            pl.BlockSpec((t_chunk, 1, b_block, E),
                         lambda b, t: (jnp.minimum(t + 1, nt - 1), b, 0, 0)),
            pl.BlockSpec((E + 8, 4 * H), lambda b, t: (0, 0)),
            pl.BlockSpec((H, 4 * H), lambda b, t: (0, 0)),
            pl.BlockSpec((H, T_pad), lambda b, t: (0, 0)),
            pl.BlockSpec((1, T_pad), lambda b, t: (0, 0)),
        ],
        out_specs=pl.BlockSpec((t_chunk, 1, b_block, T_pad),
                               lambda b, t: (t, b, 0, 0)),
        scratch_shapes=[
            pltpu.VMEM((b_block, H), jnp.bfloat16),
            pltpu.VMEM((b_block, H), jnp.float32),
            pltpu.VMEM((t_chunk * b_block, 4 * H), jnp.float32),
            pltpu.VMEM((t_chunk * b_block, H), jnp.bfloat16),
        ],
        compiler_params=pltpu.CompilerParams(
            dimension_semantics=("arbitrary", "arbitrary"),
            vmem_limit_bytes=56 * 1024 * 1024),
    )(x_v, x_v, wih_t, whh_t, wtag_t, btag)
    return out.reshape(S_pad, B_pad, T_pad)[:S, :B, :T]


# speedup vs baseline: 1.0147x; 1.0147x over previous
import jax
import jax.numpy as jnp
from jax.experimental import pallas as pl
from jax.experimental.pallas import tpu as pltpu


def _make_body(sub):
    def body(x_ref, wih_ref, whh_ref, wtag_ref, btag_ref, out_ref,
             h_ref, c_ref, gx_ref, hs_ref):
        TC, _, B, E = x_ref.shape
        H = whh_ref.shape[0]
        TP = wtag_ref.shape[1]
        NS = TC // sub

        @pl.when(pl.program_id(1) == 0)
        def _reset_state():
            h_ref[...] = jnp.zeros_like(h_ref)
            c_ref[...] = jnp.zeros_like(c_ref)

        def project_from(src_ref, j):
            xs = src_ref[pl.ds(j * sub, sub)].reshape(sub, B, E)
            chan = jax.lax.broadcasted_iota(jnp.int32, xs.shape, 2)
            xs = jnp.where(chan < 20, xs, jax.nn.sigmoid(xs)).astype(
                jnp.bfloat16)
            ones = jnp.ones((sub, B, 8), jnp.bfloat16)
            xs = jnp.concatenate([xs, ones], axis=2)
            gx_ref[pl.ds(j * sub * B, sub * B), :] = jnp.dot(
                xs.reshape(sub * B, E + 8), wih_ref[...],
                preferred_element_type=jnp.float32)

        def project_inputs(j):
            project_from(x_ref, j)

        def project_tags(j):
            tags = (jnp.dot(hs_ref[pl.ds(j * sub * B, sub * B), :],
                            wtag_ref[...],
                            preferred_element_type=jnp.float32)
                    + btag_ref[...])
            out_ref[pl.ds(j * sub, sub)] = tags.reshape(sub, 1, B, TP)

        whh = whh_ref[...]
        h = h_ref[...]
        c = c_ref[...]

        project_inputs(0)
        for j in range(NS):
            if j + 1 < NS:
                project_inputs(j + 1)
            for ts in range(sub):
                t = j * sub + ts
                z = gx_ref[pl.ds(t * B, B), :] + jnp.dot(
                    h, whh, preferred_element_type=jnp.float32)
                th = jnp.tanh(z)
                ti = th[:, 0 * H:1 * H]
                tf = th[:, 1 * H:2 * H]
                tg = th[:, 2 * H:3 * H]
                to = th[:, 3 * H:4 * H]
                c = ((tf + 1.0) * c + (ti + 1.0) * tg) * 0.5
                h = ((to + 1.0) * (0.5 * jnp.tanh(c))).astype(jnp.bfloat16)
                hs_ref[pl.ds(t * B, B), :] = h
            if j >= 1:
                project_tags(j - 1)
        project_tags(NS - 1)

        h_ref[...] = h
        c_ref[...] = c

    return body


def kernel(x, w_ih, w_hh, b_ih, b_hh, w_tag, b_tag):
    S, B, E = x.shape
    H = w_ih.shape[0] // 4
    T = w_tag.shape[0]

    B_pad = -(-B // 8) * 8
    T_pad = -(-T // 128) * 128

    b_block = B_pad
    NB = 1

    t_chunk = min(64, S)
    while S % t_chunk:
        t_chunk //= 2
    while t_chunk > 8 and t_chunk * b_block * 4 * H * 4 > 18 * 1024 * 1024:
        t_chunk //= 2
    sub = min(8, t_chunk)
    S_pad = pl.cdiv(S, t_chunk) * t_chunk

    x_p = jnp.pad(x, ((0, S_pad - S), (0, B_pad - B), (0, 0)))
    x_v = x_p.reshape(S_pad, NB, b_block, E)

    gcol = jnp.arange(4 * H)
    gate_scale = jnp.where((gcol >= 2 * H) & (gcol < 3 * H), 1.0, 0.5)

    wih_t = jnp.concatenate(
        [jnp.transpose(w_ih) * gate_scale,
         ((b_ih + b_hh) * gate_scale).reshape(1, 4 * H),
         jnp.zeros((7, 4 * H), jnp.float32)], axis=0).astype(jnp.bfloat16)
    whh_t = (jnp.transpose(w_hh) * gate_scale).astype(jnp.bfloat16)
    wtag_t = jnp.pad(jnp.transpose(w_tag).astype(jnp.bfloat16),
                     ((0, 0), (0, T_pad - T)))
    btag = jnp.pad(b_tag, (0, T_pad - T)).reshape(1, T_pad)

    nt = S_pad // t_chunk
    out = pl.pallas_call(
        _make_body(sub),
        out_shape=jax.ShapeDtypeStruct((S_pad, NB, b_block, T_pad),
                                       jnp.float32),
        grid=(NB, nt),
        in_specs=[
            pl.BlockSpec((t_chunk, 1, b_block, E), lambda b, t: (t, b, 0, 0)),
            pl.BlockSpec((E + 8, 4 * H), lambda b, t: (0, 0)),
            pl.BlockSpec((H, 4 * H), lambda b, t: (0, 0)),
            pl.BlockSpec((H, T_pad), lambda b, t: (0, 0)),
            pl.BlockSpec((1, T_pad), lambda b, t: (0, 0)),
        ],
        out_specs=pl.BlockSpec((t_chunk, 1, b_block, T_pad),
                               lambda b, t: (t, b, 0, 0)),
        scratch_shapes=[
            pltpu.VMEM((b_block, H), jnp.bfloat16),
            pltpu.VMEM((b_block, H), jnp.float32),
            pltpu.VMEM((t_chunk * b_block, 4 * H), jnp.float32),
            pltpu.VMEM((t_chunk * b_block, H), jnp.bfloat16),
        ],
        compiler_params=pltpu.CompilerParams(
            dimension_semantics=("arbitrary", "arbitrary"),
            vmem_limit_bytes=56 * 1024 * 1024),
    )(x_v, wih_t, whh_t, wtag_t, btag)
    return out.reshape(S_pad, B_pad, T_pad)[:S, :B, :T]


# speedup vs baseline: 1.0347x; 1.0197x over previous
import jax
import jax.numpy as jnp
from jax.experimental import pallas as pl
from jax.experimental.pallas import tpu as pltpu


def _make_body(sub):
    def body(x_ref, wih_ref, whh_ref, wtag_ref, btag_ref, out_ref,
             h_ref, c_ref, gx_ref, hs_ref):
        TC, _, B, E = x_ref.shape
        H = whh_ref.shape[0]
        TP = wtag_ref.shape[1]
        NS = TC // sub

        @pl.when(pl.program_id(1) == 0)
        def _reset_state():
            h_ref[...] = jnp.zeros_like(h_ref)
            c_ref[...] = jnp.zeros_like(c_ref)

        def project_from(src_ref, j):
            xs = src_ref[pl.ds(j * sub, sub)].reshape(sub, B, E)
            chan = jax.lax.broadcasted_iota(jnp.int32, xs.shape, 2)
            xs = jnp.where(chan < 20, xs, jax.nn.sigmoid(xs)).astype(
                jnp.bfloat16)
            ones = jnp.ones((sub, B, 8), jnp.bfloat16)
            xs = jnp.concatenate([xs, ones], axis=2)
            gx_ref[pl.ds(j * sub * B, sub * B), :] = jnp.dot(
                xs.reshape(sub * B, E + 8), wih_ref[...],
                preferred_element_type=jnp.float32).astype(jnp.bfloat16)

        def project_inputs(j):
            project_from(x_ref, j)

        def project_tags(j):
            tags = (jnp.dot(hs_ref[pl.ds(j * sub * B, sub * B), :],
                            wtag_ref[...],
                            preferred_element_type=jnp.float32)
                    + btag_ref[...])
            out_ref[pl.ds(j * sub, sub)] = tags.reshape(sub, 1, B, TP)

        whh = whh_ref[...]
        h = h_ref[...]
        c = c_ref[...]

        project_inputs(0)
        for j in range(NS):
            if j + 1 < NS:
                project_inputs(j + 1)
            for ts in range(sub):
                t = j * sub + ts
                z = gx_ref[pl.ds(t * B, B), :] + jnp.dot(
                    h, whh, preferred_element_type=jnp.float32)
                th = jnp.tanh(z)
                ti = th[:, 0 * H:1 * H]
                tf = th[:, 1 * H:2 * H]
                tg = th[:, 2 * H:3 * H]
                to = th[:, 3 * H:4 * H]
                c = ((tf + 1.0) * c + (ti + 1.0) * tg) * 0.5
                h = ((to + 1.0) * (0.5 * jnp.tanh(c))).astype(jnp.bfloat16)
                hs_ref[pl.ds(t * B, B), :] = h
            if j >= 1:
                project_tags(j - 1)
        project_tags(NS - 1)

        h_ref[...] = h
        c_ref[...] = c

    return body


def kernel(x, w_ih, w_hh, b_ih, b_hh, w_tag, b_tag):
    S, B, E = x.shape
    H = w_ih.shape[0] // 4
    T = w_tag.shape[0]

    B_pad = -(-B // 8) * 8
    T_pad = -(-T // 128) * 128

    b_block = B_pad
    NB = 1

    t_chunk = min(64, S)
    while S % t_chunk:
        t_chunk //= 2
    while t_chunk > 8 and t_chunk * b_block * 4 * H * 2 > 18 * 1024 * 1024:
        t_chunk //= 2
    sub = min(8, t_chunk)
    S_pad = pl.cdiv(S, t_chunk) * t_chunk

    x_p = jnp.pad(x, ((0, S_pad - S), (0, B_pad - B), (0, 0)))
    x_v = x_p.reshape(S_pad, NB, b_block, E)

    gcol = jnp.arange(4 * H)
    gate_scale = jnp.where((gcol >= 2 * H) & (gcol < 3 * H), 1.0, 0.5)

    wih_t = jnp.concatenate(
        [jnp.transpose(w_ih) * gate_scale,
         ((b_ih + b_hh) * gate_scale).reshape(1, 4 * H),
         jnp.zeros((7, 4 * H), jnp.float32)], axis=0).astype(jnp.bfloat16)
    whh_t = (jnp.transpose(w_hh) * gate_scale).astype(jnp.bfloat16)
    wtag_t = jnp.pad(jnp.transpose(w_tag).astype(jnp.bfloat16),
                     ((0, 0), (0, T_pad - T)))
    btag = jnp.pad(b_tag, (0, T_pad - T)).reshape(1, T_pad)

    nt = S_pad // t_chunk
    out = pl.pallas_call(
        _make_body(sub),
        out_shape=jax.ShapeDtypeStruct((S_pad, NB, b_block, T_pad),
                                       jnp.float32),
        grid=(NB, nt),
        in_specs=[
            pl.BlockSpec((t_chunk, 1, b_block, E), lambda b, t: (t, b, 0, 0)),
            pl.BlockSpec((E + 8, 4 * H), lambda b, t: (0, 0)),
            pl.BlockSpec((H, 4 * H), lambda b, t: (0, 0)),
            pl.BlockSpec((H, T_pad), lambda b, t: (0, 0)),
            pl.BlockSpec((1, T_pad), lambda b, t: (0, 0)),
        ],
        out_specs=pl.BlockSpec((t_chunk, 1, b_block, T_pad),
                               lambda b, t: (t, b, 0, 0)),
        scratch_shapes=[
            pltpu.VMEM((b_block, H), jnp.bfloat16),
            pltpu.VMEM((b_block, H), jnp.float32),
            pltpu.VMEM((t_chunk * b_block, 4 * H), jnp.bfloat16),
            pltpu.VMEM((t_chunk * b_block, H), jnp.bfloat16),
        ],
        compiler_params=pltpu.CompilerParams(
            dimension_semantics=("arbitrary", "arbitrary"),
            vmem_limit_bytes=56 * 1024 * 1024),
    )(x_v, wih_t, whh_t, wtag_t, btag)
    return out.reshape(S_pad, B_pad, T_pad)[:S, :B, :T]


# speedup vs baseline: 1.4245x; 1.3768x over previous
import jax
import jax.numpy as jnp
from jax.experimental import pallas as pl
from jax.experimental.pallas import tpu as pltpu


def _make_body(sub):
    def body(x_ref, wih_ref, whh_ref, wtag_ref, btag_ref, out_ref,
             h_ref, c_ref, gx_ref, hs_ref):
        TC, _, B, E = x_ref.shape
        H = whh_ref.shape[0]
        TP = wtag_ref.shape[1]
        NS = TC // sub

        @pl.when(pl.program_id(1) == 0)
        def _reset_state():
            h_ref[...] = jnp.zeros_like(h_ref)
            c_ref[...] = jnp.zeros_like(c_ref)

        def project_from(src_ref, j):
            xs = src_ref[pl.ds(j * sub, sub)].reshape(sub, B, E)
            chan = jax.lax.broadcasted_iota(jnp.int32, xs.shape, 2)
            xs = jnp.where(chan < 20, xs, jax.nn.sigmoid(xs)).astype(
                jnp.bfloat16)
            ones = jnp.ones((sub, B, 8), jnp.bfloat16)
            xs = jnp.concatenate([xs, ones], axis=2)
            gx_ref[pl.ds(j * sub * B, sub * B), :] = jnp.dot(
                xs.reshape(sub * B, E + 8), wih_ref[...],
                preferred_element_type=jnp.float32).astype(jnp.bfloat16)

        def project_inputs(j):
            project_from(x_ref, j)

        def project_tags(j):
            tags = (jnp.dot(hs_ref[pl.ds(j * sub * B, sub * B), :],
                            wtag_ref[...],
                            preferred_element_type=jnp.float32)
                    + btag_ref[...])
            out_ref[pl.ds(j * sub, sub)] = tags.reshape(sub, 1, B, TP)

        whh = whh_ref[...]
        h = h_ref[...]
        c = c_ref[...]

        project_inputs(0)
        for j in range(NS):
            if j + 1 < NS:
                project_inputs(j + 1)
            for ts in range(sub):
                t = j * sub + ts
                z = gx_ref[pl.ds(t * B, B), :] + jnp.dot(
                    h, whh, preferred_element_type=jnp.float32)
                th = jnp.tanh(z)
                ti = th[:, 0 * H:1 * H]
                tf = th[:, 1 * H:2 * H]
                tg = th[:, 2 * H:3 * H]
                to = th[:, 3 * H:4 * H]
                c = ((tf + 1.0) * c + (ti + 1.0) * tg) * 0.5
                h = ((to + 1.0) * (0.5 * jnp.tanh(c))).astype(jnp.bfloat16)
                hs_ref[pl.ds(t * B, B), :] = h
            if j >= 1:
                project_tags(j - 1)
        project_tags(NS - 1)

        h_ref[...] = h
        c_ref[...] = c

    return body


def kernel(x, w_ih, w_hh, b_ih, b_hh, w_tag, b_tag):
    S, B, E = x.shape
    H = w_ih.shape[0] // 4
    T = w_tag.shape[0]

    B_pad = -(-B // 8) * 8
    T_pad = -(-T // 128) * 128

    b_block = B_pad
    NB = 1

    t_chunk = min(64, S)
    while S % t_chunk:
        t_chunk //= 2
    while t_chunk > 8 and t_chunk * b_block * 4 * H * 2 > 18 * 1024 * 1024:
        t_chunk //= 2
    sub = min(1, t_chunk)
    S_pad = pl.cdiv(S, t_chunk) * t_chunk

    x_p = jnp.pad(x, ((0, S_pad - S), (0, B_pad - B), (0, 0)))
    x_v = x_p.reshape(S_pad, NB, b_block, E)

    gcol = jnp.arange(4 * H)
    gate_scale = jnp.where((gcol >= 2 * H) & (gcol < 3 * H), 1.0, 0.5)

    wih_t = jnp.concatenate(
        [jnp.transpose(w_ih) * gate_scale,
         ((b_ih + b_hh) * gate_scale).reshape(1, 4 * H),
         jnp.zeros((7, 4 * H), jnp.float32)], axis=0).astype(jnp.bfloat16)
    whh_t = (jnp.transpose(w_hh) * gate_scale).astype(jnp.bfloat16)
    wtag_t = jnp.pad(jnp.transpose(w_tag).astype(jnp.bfloat16),
                     ((0, 0), (0, T_pad - T)))
    btag = jnp.pad(b_tag, (0, T_pad - T)).reshape(1, T_pad)

    nt = S_pad // t_chunk
    out = pl.pallas_call(
        _make_body(sub),
        out_shape=jax.ShapeDtypeStruct((S_pad, NB, b_block, T_pad),
                                       jnp.float32),
        grid=(NB, nt),
        in_specs=[
            pl.BlockSpec((t_chunk, 1, b_block, E), lambda b, t: (t, b, 0, 0)),
            pl.BlockSpec((E + 8, 4 * H), lambda b, t: (0, 0)),
            pl.BlockSpec((H, 4 * H), lambda b, t: (0, 0)),
            pl.BlockSpec((H, T_pad), lambda b, t: (0, 0)),
            pl.BlockSpec((1, T_pad), lambda b, t: (0, 0)),
        ],
        out_specs=pl.BlockSpec((t_chunk, 1, b_block, T_pad),
                               lambda b, t: (t, b, 0, 0)),
        scratch_shapes=[
            pltpu.VMEM((b_block, H), jnp.bfloat16),
            pltpu.VMEM((b_block, H), jnp.float32),
            pltpu.VMEM((t_chunk * b_block, 4 * H), jnp.bfloat16),
            pltpu.VMEM((t_chunk * b_block, H), jnp.bfloat16),
        ],
        compiler_params=pltpu.CompilerParams(
            dimension_semantics=("arbitrary", "arbitrary"),
            vmem_limit_bytes=56 * 1024 * 1024),
    )(x_v, wih_t, whh_t, wtag_t, btag)
    return out.reshape(S_pad, B_pad, T_pad)[:S, :B, :T]


# speedup vs baseline: 1.4333x; 1.0061x over previous
import jax
import jax.numpy as jnp
from jax.experimental import pallas as pl
from jax.experimental.pallas import tpu as pltpu


def _make_body(sub):
    def body(x_ref, wih_ref, whh_ref, wtag_ref, btag_ref, out_ref,
             h_ref, c_ref, gx_ref, hs_ref):
        TC, _, B, E = x_ref.shape
        H = whh_ref.shape[0]
        TP = wtag_ref.shape[1]
        NS = TC // sub

        @pl.when(pl.program_id(1) == 0)
        def _reset_state():
            h_ref[...] = jnp.zeros_like(h_ref)
            c_ref[...] = jnp.zeros_like(c_ref)

        def project_from(src_ref, j):
            xs = src_ref[pl.ds(j * sub, sub)].reshape(sub, B, E)
            chan = jax.lax.broadcasted_iota(jnp.int32, xs.shape, 2)
            xs = jnp.where(chan < 20, xs, jax.nn.sigmoid(xs)).astype(
                jnp.bfloat16)
            ones = jnp.ones((sub, B, 8), jnp.bfloat16)
            xs = jnp.concatenate([xs, ones], axis=2)
            gx_ref[pl.ds(j * sub * B, sub * B), :] = jnp.dot(
                xs.reshape(sub * B, E + 8), wih_ref[...],
                preferred_element_type=jnp.float32).astype(jnp.bfloat16)

        def project_inputs(j):
            project_from(x_ref, j)

        def project_tags(j):
            TB = min(sub, TC)
            tags = (jnp.dot(hs_ref[pl.ds(j * TB * B, TB * B), :],
                            wtag_ref[...],
                            preferred_element_type=jnp.float32)
                    + btag_ref[...])
            out_ref[pl.ds(j * TB, TB)] = tags.reshape(TB, 1, B, TP)

        whh = whh_ref[...]
        h = h_ref[...]
        c = c_ref[...]

        TB = min(sub, TC)
        NT = TC // TB

        project_inputs(0)
        for j in range(NS):
            if j + 1 < NS:
                project_inputs(j + 1)
            for ts in range(sub):
                t = j * sub + ts
                z = gx_ref[pl.ds(t * B, B), :] + jnp.dot(
                    h, whh, preferred_element_type=jnp.float32)
                th = jnp.tanh(z)
                ti = th[:, 0 * H:1 * H]
                tf = th[:, 1 * H:2 * H]
                tg = th[:, 2 * H:3 * H]
                to = th[:, 3 * H:4 * H]
                c = ((tf + 1.0) * c + (ti + 1.0) * tg) * 0.5
                h = ((to + 1.0) * jnp.tanh(c)).astype(jnp.bfloat16)
                hs_ref[pl.ds(t * B, B), :] = h
                if t % TB == TB - 1 and t >= TB:
                    project_tags(t // TB - 1)
        project_tags(NT - 1)

        h_ref[...] = h
        c_ref[...] = c

    return body


def kernel(x, w_ih, w_hh, b_ih, b_hh, w_tag, b_tag):
    S, B, E = x.shape
    H = w_ih.shape[0] // 4
    T = w_tag.shape[0]

    B_pad = -(-B // 8) * 8
    T_pad = -(-T // 128) * 128

    b_block = B_pad
    NB = 1

    t_chunk = min(64, S)
    while S % t_chunk:
        t_chunk //= 2
    while t_chunk > 8 and t_chunk * b_block * 4 * H * 2 > 18 * 1024 * 1024:
        t_chunk //= 2
    sub = min(1, t_chunk)
    S_pad = pl.cdiv(S, t_chunk) * t_chunk

    x_p = jnp.pad(x, ((0, S_pad - S), (0, B_pad - B), (0, 0)))
    x_v = x_p.reshape(S_pad, NB, b_block, E)

    gcol = jnp.arange(4 * H)
    gate_scale = jnp.where((gcol >= 2 * H) & (gcol < 3 * H), 1.0, 0.5)

    wih_t = jnp.concatenate(
        [jnp.transpose(w_ih) * gate_scale,
         ((b_ih + b_hh) * gate_scale).reshape(1, 4 * H),
         jnp.zeros((7, 4 * H), jnp.float32)], axis=0).astype(jnp.bfloat16)
    whh_t = (jnp.transpose(w_hh) * 0.5 * gate_scale).astype(jnp.bfloat16)
    wtag_t = jnp.pad((jnp.transpose(w_tag) * 0.5).astype(jnp.bfloat16),
                     ((0, 0), (0, T_pad - T)))
    btag = jnp.pad(b_tag, (0, T_pad - T)).reshape(1, T_pad)

    nt = S_pad // t_chunk
    out = pl.pallas_call(
        _make_body(sub),
        out_shape=jax.ShapeDtypeStruct((S_pad, NB, b_block, T_pad),
                                       jnp.float32),
        grid=(NB, nt),
        in_specs=[
            pl.BlockSpec((t_chunk, 1, b_block, E), lambda b, t: (t, b, 0, 0)),
            pl.BlockSpec((E + 8, 4 * H), lambda b, t: (0, 0)),
            pl.BlockSpec((H, 4 * H), lambda b, t: (0, 0)),
            pl.BlockSpec((H, T_pad), lambda b, t: (0, 0)),
            pl.BlockSpec((1, T_pad), lambda b, t: (0, 0)),
        ],
        out_specs=pl.BlockSpec((t_chunk, 1, b_block, T_pad),
                               lambda b, t: (t, b, 0, 0)),
        scratch_shapes=[
            pltpu.VMEM((b_block, H), jnp.bfloat16),
            pltpu.VMEM((b_block, H), jnp.float32),
            pltpu.VMEM((t_chunk * b_block, 4 * H), jnp.bfloat16),
            pltpu.VMEM((t_chunk * b_block, H), jnp.bfloat16),
        ],
        compiler_params=pltpu.CompilerParams(
            dimension_semantics=("arbitrary", "arbitrary"),
            vmem_limit_bytes=56 * 1024 * 1024),
    )(x_v, wih_t, whh_t, wtag_t, btag)
    return out.reshape(S_pad, B_pad, T_pad)[:S, :B, :T]
